# baseline (speedup 1.0000x reference)
import functools

import jax
import jax.numpy as jnp
from jax import lax
from jax.experimental import pallas as pl
from jax.experimental.pallas import tpu as pltpu
from jax.experimental.pallas import tpu_sc as plsc

N = 10000
E = 160000
NP = 10240
HALF = NP // 2
CH = HALF // 2
TRASH = 16
ACCR = CH + TRASH
CB = 128
K = 128
KS = 256
NSUB = 16
NCORE = 2
EW_DEG = E // (NSUB * NCORE)
EW_SC = E // NSUB
NDEG = 10496
DSTR = NDEG // NSUB
RB = 512
LCAP = EW_SC + 2 * KS + 16

_SC_PARAMS = pltpu.CompilerParams(needs_layout_passes=False)


def _mesh():
    return plsc.VectorSubcoreMesh(core_axis_name="c", subcore_axis_name="s")


def _deg_body(src_hbm, dst_hbm, out_hbm, srcv, dstv, lst, cur, ones_v, zb,
              tmp, deg_sh, sem):
    cid = lax.axis_index("c")
    sid = lax.axis_index("s")
    z = jnp.zeros((16,), jnp.float32)
    for u in range(DSTR // 16):
        zb[pl.ds(u * 16, 16)] = z
    for t in range(NSUB):
        pltpu.sync_copy(zb, deg_sh.at[pl.ds(sid * NDEG + t * DSTR, DSTR)])
    one = jnp.ones((16,), jnp.float32)
    for u in range(K // 16):
        ones_v[pl.ds(u * 16, 16)] = one
    wid = cid * NSUB + sid
    pltpu.sync_copy(src_hbm.at[pl.ds(wid * EW_DEG, EW_DEG)],
                    srcv.at[pl.ds(0, EW_DEG)])
    pltpu.sync_copy(dst_hbm.at[pl.ds(wid * EW_DEG, EW_DEG)],
                    dstv.at[pl.ds(0, EW_DEG)])

    lane = lax.iota(jnp.int32, 16)

    def step(i, cnt):
        s16 = srcv[pl.ds(i * 16, 16)]
        d16 = dstv[pl.ds(i * 16, 16)]
        m = (s16 != d16) & (i * 16 + lane < EW_DEG)
        plsc.store_compressed(lst.at[pl.ds(cnt, 16)], s16, mask=m)
        return cnt + jnp.sum(m.astype(jnp.int32))

    cnt = lax.fori_loop(0, (EW_DEG + 15) // 16, step, jnp.int32(0))
    pad = NP + sid * 16 + lax.iota(jnp.int32, 16)
    for j in range(K // 16):
        lst[pl.ds(cnt + j * 16, 16)] = pad
    nblk = (cnt + K - 1) // K
    base = sid * NDEG

    def flush(j, c):
        for u in range(K // 16):
            cur[pl.ds(u * 16, 16)] = lst[pl.ds(j * K + u * 16, 16)] + base
        pltpu.sync_copy(ones_v, deg_sh.at[cur], add=True)
        return c

    lax.fori_loop(0, nblk, flush, jnp.int32(0))
    plsc.subcore_barrier()
    for u in range(DSTR // 16):
        zb[pl.ds(u * 16, 16)] = z
    for w in range(NSUB):
        pltpu.sync_copy(deg_sh.at[pl.ds(w * NDEG + sid * DSTR, DSTR)], tmp)
        for u in range(DSTR // 16):
            zb[pl.ds(u * 16, 16)] = zb[pl.ds(u * 16, 16)] + tmp[pl.ds(u * 16, 16)]
    pltpu.sync_copy(zb, out_hbm.at[pl.ds(cid * NDEG + sid * DSTR, DSTR)])


_deg_call = pl.kernel(
    _deg_body,
    out_type=jax.ShapeDtypeStruct((NCORE * NDEG,), jnp.float32),
    mesh=_mesh(),
    compiler_params=_SC_PARAMS,
    scratch_types=[
        pltpu.VMEM((EW_DEG + 16,), jnp.int32),
        pltpu.VMEM((EW_DEG + 16,), jnp.int32),
        pltpu.VMEM((EW_DEG + 2 * K + 16,), jnp.int32),
        pltpu.VMEM((K,), jnp.int32),
        pltpu.VMEM((K,), jnp.float32),
        pltpu.VMEM((DSTR,), jnp.float32),
        pltpu.VMEM((DSTR,), jnp.float32),
        pltpu.VMEM_SHARED((NSUB * NDEG,), jnp.float32),
        pltpu.SemaphoreType.DMA,
    ],
    name="gcn_degree_sc",
)


def _scat_body(NB, src_hbm, dst_hbm, g_hbm, agg_hbm, srcv, dstv,
               lsrc, ldst, cs0, cd0, cs1, cd1, st0, st1, zb, acc, sm0, sm1):
    cid = lax.axis_index("c")
    sid = lax.axis_index("s")
    lo = cid * HALF
    z = jnp.zeros((16,), jnp.float32)
    for r in range(16):
        for u in range(CB // 16):
            zb[r, pl.ds(u * 16, 16)] = z
    pltpu.sync_copy(src_hbm.at[pl.ds(sid * EW_SC, EW_SC)], srcv)
    pltpu.sync_copy(dst_hbm.at[pl.ds(sid * EW_SC, EW_SC)], dstv)
    pad_src = sid * 16 + lax.iota(jnp.int32, 16)
    pad_dst = CH + lax.iota(jnp.int32, 16)
    bufs = ((cs0, cd0, st0, sm0), (cs1, cd1, st1, sm1))

    for chunk in range(2):
        b0 = lo + chunk * CH

        def step(i, cnt):
            s16 = srcv[pl.ds(i * 16, 16)]
            d16 = dstv[pl.ds(i * 16, 16)]
            rel = d16 - b0
            m = (rel >= 0) & (rel < CH) & (s16 != d16)
            plsc.store_compressed(lsrc.at[pl.ds(cnt, 16)], s16, mask=m)
            plsc.store_compressed(ldst.at[pl.ds(cnt, 16)], rel, mask=m)
            return cnt + jnp.sum(m.astype(jnp.int32))

        cnt = lax.fori_loop(0, EW_SC // 16, step, jnp.int32(0))
        for j in range(2 * KS // 16):
            lsrc[pl.ds(cnt + j * 16, 16)] = pad_src
            ldst[pl.ds(cnt + j * 16, 16)] = pad_dst
        ngrp = (cnt + 2 * KS - 1) // (2 * KS)

        for cb in range(NB):
            zb_base = sid * (ACCR // NSUB)
            for t in range(10):
                pltpu.sync_copy(zb, acc.at[pl.ds(zb_base + t * 16, 16)])
            pltpu.sync_copy(zb.at[pl.ds(0, 1)],
                            acc.at[pl.ds(zb_base + 160, 1)])
            plsc.subcore_barrier()

            def flush2(jj, c):
                ds = []
                for q in range(2):
                    csrc, cdst, stage, gsem = bufs[q]
                    j = 2 * jj + q
                    for u in range(KS // 16):
                        csrc[pl.ds(u * 16, 16)] = \
                            lsrc[pl.ds(j * KS + u * 16, 16)]
                        cdst[pl.ds(u * 16, 16)] = \
                            ldst[pl.ds(j * KS + u * 16, 16)]
                    ds.append(pltpu.async_copy(g_hbm.at[cb].at[csrc],
                                               stage, gsem))
                for q in range(2):
                    _, cdst, stage, _ = bufs[q]
                    ds[q].wait()
                    pltpu.sync_copy(stage, acc.at[cdst], add=True)
                return c

            lax.fori_loop(0, ngrp, flush2, jnp.int32(0))
            plsc.subcore_barrier()
            ob = sid * (CH // NSUB)
            row0 = lo + chunk * CH + ob
            for t in range(2):
                pltpu.sync_copy(acc.at[pl.ds(ob + t * 80, 80)],
                                st0.at[pl.ds(0, 80)])
                pltpu.sync_copy(st0.at[pl.ds(0, 80)],
                                agg_hbm.at[cb, pl.ds(row0 + t * 80, 80)])
            plsc.subcore_barrier()


def _make_scatter(NB):
    return pl.kernel(
        functools.partial(_scat_body, NB),
        out_type=jax.ShapeDtypeStruct((NB, NP, CB), jnp.float32),
        mesh=_mesh(),
        compiler_params=_SC_PARAMS,
        scratch_types=[
            pltpu.VMEM((EW_SC,), jnp.int32),
            pltpu.VMEM((EW_SC,), jnp.int32),
            pltpu.VMEM((LCAP,), jnp.int32),
            pltpu.VMEM((LCAP,), jnp.int32),
        ] + [pltpu.VMEM((KS,), jnp.int32)] * 4 + [
            pltpu.VMEM((KS, CB), jnp.float32),
            pltpu.VMEM((KS, CB), jnp.float32),
            pltpu.VMEM((16, CB), jnp.float32),
            pltpu.VMEM_SHARED((ACCR, CB), jnp.float32),
            pltpu.SemaphoreType.DMA,
            pltpu.SemaphoreType.DMA,
        ],
        name=f"gcn_scatter_sc_{NB}",
    )


_scatter_4 = _make_scatter(4)
_scatter_2 = _make_scatter(2)


def _mm1_body(x_ref, w_ref, deg_ref, g_ref):
    s = lax.rsqrt(deg_ref[0, :] + deg_ref[1, :] + 1.0)
    h = jnp.dot(x_ref[...], w_ref[...], preferred_element_type=jnp.float32)
    h = h * s[:, None]
    for cb in range(g_ref.shape[0]):
        g_ref[cb] = h[:, cb * CB:(cb + 1) * CB]


def _mid_body(agg_ref, g_ref, deg_ref, b_ref, w_ref, out_ref):
    s = lax.rsqrt(deg_ref[0, :] + deg_ref[1, :] + 1.0)
    nbi = agg_ref.shape[0]
    t = jnp.concatenate(
        [agg_ref[cb] + g_ref[cb] for cb in range(nbi)], axis=1)
    t = jnp.maximum(s[:, None] * t + b_ref[...], 0.0)
    h = jnp.dot(t, w_ref[...], preferred_element_type=jnp.float32)
    h = h * s[:, None]
    for cb in range(out_ref.shape[0]):
        out_ref[cb] = h[:, cb * CB:(cb + 1) * CB]


def _fin_body(agg_ref, g_ref, deg_ref, b_ref, out_ref):
    s = lax.rsqrt(deg_ref[0, :] + deg_ref[1, :] + 1.0)
    nbi = agg_ref.shape[0]
    t = jnp.concatenate(
        [agg_ref[cb] + g_ref[cb] for cb in range(nbi)], axis=1)
    o = s[:, None] * t + b_ref[...]
    denom = jnp.maximum(jnp.sum(jnp.abs(o), axis=1, keepdims=True), 1e-12)
    out_ref[...] = o / denom


def _blk3(nb):
    return pl.BlockSpec((nb, RB, CB), lambda i: (0, i, 0))


def _mm1(xp, W1, deg2):
    din, dout = W1.shape
    nb = dout // CB
    return pl.pallas_call(
        _mm1_body,
        grid=(NP // RB,),
        in_specs=[
            pl.BlockSpec((RB, din), lambda i: (i, 0)),
            pl.BlockSpec((din, dout), lambda i: (0, 0)),
            pl.BlockSpec((2, RB), lambda i: (0, i)),
        ],
        out_specs=_blk3(nb),
        out_shape=jax.ShapeDtypeStruct((nb, NP, CB), jnp.float32),
    )(xp, W1, deg2)


def _mid(agg, g, deg2, b, W):
    din, dout = W.shape
    nbi, nbo = din // CB, dout // CB
    return pl.pallas_call(
        _mid_body,
        grid=(NP // RB,),
        in_specs=[
            _blk3(nbi),
            _blk3(nbi),
            pl.BlockSpec((2, RB), lambda i: (0, i)),
            pl.BlockSpec((1, din), lambda i: (0, 0)),
            pl.BlockSpec((din, dout), lambda i: (0, 0)),
        ],
        out_specs=_blk3(nbo),
        out_shape=jax.ShapeDtypeStruct((nbo, NP, CB), jnp.float32),
    )(agg, g, deg2, b, W)


def _fin(agg, g, deg2, b):
    nbi = agg.shape[0]
    d = nbi * CB
    return pl.pallas_call(
        _fin_body,
        grid=(NP // RB,),
        in_specs=[
            _blk3(nbi),
            _blk3(nbi),
            pl.BlockSpec((2, RB), lambda i: (0, i)),
            pl.BlockSpec((1, d), lambda i: (0, 0)),
        ],
        out_specs=pl.BlockSpec((RB, d), lambda i: (i, 0)),
        out_shape=jax.ShapeDtypeStruct((NP, d), jnp.float32),
    )(agg, g, deg2, b)


def kernel(x, edge_index, W1, b1, W2, b2, WL, bL):
    ei = edge_index.astype(jnp.int32)
    src, dst = ei[0], ei[1]
    xp = jnp.pad(x, ((0, NP - N), (0, 0)))
    deg2 = _deg_call(src, dst).reshape(NCORE, NDEG)[:, :NP]
    g1 = _mm1(xp, W1, deg2)
    agg1 = _scatter_4(src, dst, g1)
    g2 = _mid(agg1, g1, deg2, b1.reshape(1, -1), W2)
    agg2 = _scatter_4(src, dst, g2)
    g3 = _mid(agg2, g2, deg2, b2.reshape(1, -1), WL)
    agg3 = _scatter_2(src, dst, g3)
    out = _fin(agg3, g3, deg2, bL.reshape(1, -1))
    return out[:N]


# speedup vs baseline: 1.1683x; 1.1683x over previous
import functools

import jax
import jax.numpy as jnp
from jax import lax
from jax.experimental import pallas as pl
from jax.experimental.pallas import tpu as pltpu
from jax.experimental.pallas import tpu_sc as plsc

N = 10000
E = 160000
NP = 10240
HALF = NP // 2
CH = HALF // 2
TRASH = 16
ACCR = CH + TRASH
CB = 128
K = 128
KS = 128
NSUB = 16
NCORE = 2
EW_DEG = E // (NSUB * NCORE)
EW_SC = E // NSUB
NDEG = 10496
DSTR = NDEG // NSUB
RB = 512
LCAP = EW_SC + 2 * KS + 16

_SC_PARAMS = pltpu.CompilerParams(needs_layout_passes=False)


def _mesh():
    return plsc.VectorSubcoreMesh(core_axis_name="c", subcore_axis_name="s")


def _deg_body(src_hbm, dst_hbm, out_hbm, srcv, dstv, lst, cur, ones_v, zb,
              tmp, deg_sh, sem):
    cid = lax.axis_index("c")
    sid = lax.axis_index("s")
    z = jnp.zeros((16,), jnp.float32)
    for u in range(DSTR // 16):
        zb[pl.ds(u * 16, 16)] = z
    for t in range(NSUB):
        pltpu.sync_copy(zb, deg_sh.at[pl.ds(sid * NDEG + t * DSTR, DSTR)])
    one = jnp.ones((16,), jnp.float32)
    for u in range(K // 16):
        ones_v[pl.ds(u * 16, 16)] = one
    wid = cid * NSUB + sid
    pltpu.sync_copy(src_hbm.at[pl.ds(wid * EW_DEG, EW_DEG)],
                    srcv.at[pl.ds(0, EW_DEG)])
    pltpu.sync_copy(dst_hbm.at[pl.ds(wid * EW_DEG, EW_DEG)],
                    dstv.at[pl.ds(0, EW_DEG)])

    lane = lax.iota(jnp.int32, 16)

    def step(i, cnt):
        s16 = srcv[pl.ds(i * 16, 16)]
        d16 = dstv[pl.ds(i * 16, 16)]
        m = (s16 != d16) & (i * 16 + lane < EW_DEG)
        plsc.store_compressed(lst.at[pl.ds(cnt, 16)], s16, mask=m)
        return cnt + jnp.sum(m.astype(jnp.int32))

    cnt = lax.fori_loop(0, (EW_DEG + 15) // 16, step, jnp.int32(0))
    pad = NP + sid * 16 + lax.iota(jnp.int32, 16)
    for j in range(K // 16):
        lst[pl.ds(cnt + j * 16, 16)] = pad
    nblk = (cnt + K - 1) // K
    base = sid * NDEG

    def flush(j, c):
        for u in range(K // 16):
            cur[pl.ds(u * 16, 16)] = lst[pl.ds(j * K + u * 16, 16)] + base
        pltpu.sync_copy(ones_v, deg_sh.at[cur], add=True)
        return c

    lax.fori_loop(0, nblk, flush, jnp.int32(0))
    plsc.subcore_barrier()
    for u in range(DSTR // 16):
        zb[pl.ds(u * 16, 16)] = z
    for w in range(NSUB):
        pltpu.sync_copy(deg_sh.at[pl.ds(w * NDEG + sid * DSTR, DSTR)], tmp)
        for u in range(DSTR // 16):
            zb[pl.ds(u * 16, 16)] = zb[pl.ds(u * 16, 16)] + tmp[pl.ds(u * 16, 16)]
    pltpu.sync_copy(zb, out_hbm.at[pl.ds(cid * NDEG + sid * DSTR, DSTR)])


_deg_call = pl.kernel(
    _deg_body,
    out_type=jax.ShapeDtypeStruct((NCORE * NDEG,), jnp.float32),
    mesh=_mesh(),
    compiler_params=_SC_PARAMS,
    scratch_types=[
        pltpu.VMEM((EW_DEG + 16,), jnp.int32),
        pltpu.VMEM((EW_DEG + 16,), jnp.int32),
        pltpu.VMEM((EW_DEG + 2 * K + 16,), jnp.int32),
        pltpu.VMEM((K,), jnp.int32),
        pltpu.VMEM((K,), jnp.float32),
        pltpu.VMEM((DSTR,), jnp.float32),
        pltpu.VMEM((DSTR,), jnp.float32),
        pltpu.VMEM_SHARED((NSUB * NDEG,), jnp.float32),
        pltpu.SemaphoreType.DMA,
    ],
    name="gcn_degree_sc",
)


def _scat_body(NB, src_hbm, dst_hbm, g_hbm, agg_hbm, srcv, dstv,
               lsrc, ldst, cs0, cd0, cs1, cd1, st0, st1, zb, acc, sm0, sm1):
    cid = lax.axis_index("c")
    sid = lax.axis_index("s")
    lo = cid * HALF
    z = jnp.zeros((16,), jnp.float32)
    for r in range(16):
        for u in range(CB // 16):
            zb[r, pl.ds(u * 16, 16)] = z
    pltpu.sync_copy(src_hbm.at[pl.ds(sid * EW_SC, EW_SC)], srcv)
    pltpu.sync_copy(dst_hbm.at[pl.ds(sid * EW_SC, EW_SC)], dstv)
    pad_src = sid * 16 + lax.iota(jnp.int32, 16)
    pad_dst = CH + lax.iota(jnp.int32, 16)
    bufs = ((cs0, cd0, st0, sm0), (cs1, cd1, st1, sm1))

    for chunk in range(2):
        b0 = lo + chunk * CH

        def step(i, cnt):
            s16 = srcv[pl.ds(i * 16, 16)]
            d16 = dstv[pl.ds(i * 16, 16)]
            rel = d16 - b0
            m = (rel >= 0) & (rel < CH) & (s16 != d16)
            plsc.store_compressed(lsrc.at[pl.ds(cnt, 16)], s16, mask=m)
            plsc.store_compressed(ldst.at[pl.ds(cnt, 16)], rel, mask=m)
            return cnt + jnp.sum(m.astype(jnp.int32))

        cnt = lax.fori_loop(0, EW_SC // 16, step, jnp.int32(0))
        for j in range(2 * KS // 16):
            lsrc[pl.ds(cnt + j * 16, 16)] = pad_src
            ldst[pl.ds(cnt + j * 16, 16)] = pad_dst
        ngrp = (cnt + 2 * KS - 1) // (2 * KS)

        def build(csrc, cdst, j):
            for u in range(KS // 16):
                csrc[pl.ds(u * 16, 16)] = lsrc[pl.ds(j * KS + u * 16, 16)]
                cdst[pl.ds(u * 16, 16)] = ldst[pl.ds(j * KS + u * 16, 16)]

        for half in range(NB // 2):
            cb0, cb1 = 2 * half, 2 * half + 1
            zb_base = sid * (ACCR // NSUB)
            for p in range(2):
                for t in range(10):
                    pltpu.sync_copy(zb, acc.at[p, pl.ds(zb_base + t * 16, 16)])
                pltpu.sync_copy(zb.at[pl.ds(0, 1)],
                                acc.at[p, pl.ds(zb_base + 160, 1)])
            plsc.subcore_barrier()

            def flush2(jj, c):
                j0 = 2 * jj
                build(cs0, cd0, j0)
                d0 = pltpu.async_copy(g_hbm.at[cb0].at[cs0], st0, sm0)
                d1 = pltpu.async_copy(g_hbm.at[cb1].at[cs0], st1, sm1)
                build(cs1, cd1, j0 + 1)
                d0.wait()
                pltpu.sync_copy(st0, acc.at[0].at[cd0], add=True)
                d2 = pltpu.async_copy(g_hbm.at[cb0].at[cs1], st0, sm0)
                d1.wait()
                pltpu.sync_copy(st1, acc.at[1].at[cd0], add=True)
                d3 = pltpu.async_copy(g_hbm.at[cb1].at[cs1], st1, sm1)
                d2.wait()
                pltpu.sync_copy(st0, acc.at[0].at[cd1], add=True)
                d3.wait()
                pltpu.sync_copy(st1, acc.at[1].at[cd1], add=True)
                return c

            lax.fori_loop(0, ngrp, flush2, jnp.int32(0))
            plsc.subcore_barrier()
            ob = sid * (CH // NSUB)
            row0 = lo + chunk * CH + ob
            for p, cb in ((0, cb0), (1, cb1)):
                for t in range(2):
                    pltpu.sync_copy(acc.at[p, pl.ds(ob + t * 80, 80)],
                                    st0.at[pl.ds(0, 80)])
                    pltpu.sync_copy(st0.at[pl.ds(0, 80)],
                                    agg_hbm.at[cb, pl.ds(row0 + t * 80, 80)])
            plsc.subcore_barrier()


def _make_scatter(NB):
    return pl.kernel(
        functools.partial(_scat_body, NB),
        out_type=jax.ShapeDtypeStruct((NB, NP, CB), jnp.float32),
        mesh=_mesh(),
        compiler_params=_SC_PARAMS,
        scratch_types=[
            pltpu.VMEM((EW_SC,), jnp.int32),
            pltpu.VMEM((EW_SC,), jnp.int32),
            pltpu.VMEM((LCAP,), jnp.int32),
            pltpu.VMEM((LCAP,), jnp.int32),
        ] + [pltpu.VMEM((KS,), jnp.int32)] * 4 + [
            pltpu.VMEM((KS, CB), jnp.float32),
            pltpu.VMEM((KS, CB), jnp.float32),
            pltpu.VMEM((16, CB), jnp.float32),
            pltpu.VMEM_SHARED((2, ACCR, CB), jnp.float32),
            pltpu.SemaphoreType.DMA,
            pltpu.SemaphoreType.DMA,
        ],
        name=f"gcn_scatter_sc_{NB}",
    )


_scatter_4 = _make_scatter(4)
_scatter_2 = _make_scatter(2)


def _mm1_body(x_ref, w_ref, deg_ref, g_ref):
    s = lax.rsqrt(deg_ref[0, :] + deg_ref[1, :] + 1.0)
    h = jnp.dot(x_ref[...], w_ref[...], preferred_element_type=jnp.float32)
    h = h * s[:, None]
    for cb in range(g_ref.shape[0]):
        g_ref[cb] = h[:, cb * CB:(cb + 1) * CB]


def _mid_body(agg_ref, g_ref, deg_ref, b_ref, w_ref, out_ref):
    s = lax.rsqrt(deg_ref[0, :] + deg_ref[1, :] + 1.0)
    nbi = agg_ref.shape[0]
    t = jnp.concatenate(
        [agg_ref[cb] + g_ref[cb] for cb in range(nbi)], axis=1)
    t = jnp.maximum(s[:, None] * t + b_ref[...], 0.0)
    h = jnp.dot(t, w_ref[...], preferred_element_type=jnp.float32)
    h = h * s[:, None]
    for cb in range(out_ref.shape[0]):
        out_ref[cb] = h[:, cb * CB:(cb + 1) * CB]


def _fin_body(agg_ref, g_ref, deg_ref, b_ref, out_ref):
    s = lax.rsqrt(deg_ref[0, :] + deg_ref[1, :] + 1.0)
    nbi = agg_ref.shape[0]
    t = jnp.concatenate(
        [agg_ref[cb] + g_ref[cb] for cb in range(nbi)], axis=1)
    o = s[:, None] * t + b_ref[...]
    denom = jnp.maximum(jnp.sum(jnp.abs(o), axis=1, keepdims=True), 1e-12)
    out_ref[...] = o / denom


def _blk3(nb):
    return pl.BlockSpec((nb, RB, CB), lambda i: (0, i, 0))


def _mm1(xp, W1, deg2):
    din, dout = W1.shape
    nb = dout // CB
    return pl.pallas_call(
        _mm1_body,
        grid=(NP // RB,),
        in_specs=[
            pl.BlockSpec((RB, din), lambda i: (i, 0)),
            pl.BlockSpec((din, dout), lambda i: (0, 0)),
            pl.BlockSpec((2, RB), lambda i: (0, i)),
        ],
        out_specs=_blk3(nb),
        out_shape=jax.ShapeDtypeStruct((nb, NP, CB), jnp.float32),
    )(xp, W1, deg2)


def _mid(agg, g, deg2, b, W):
    din, dout = W.shape
    nbi, nbo = din // CB, dout // CB
    return pl.pallas_call(
        _mid_body,
        grid=(NP // RB,),
        in_specs=[
            _blk3(nbi),
            _blk3(nbi),
            pl.BlockSpec((2, RB), lambda i: (0, i)),
            pl.BlockSpec((1, din), lambda i: (0, 0)),
            pl.BlockSpec((din, dout), lambda i: (0, 0)),
        ],
        out_specs=_blk3(nbo),
        out_shape=jax.ShapeDtypeStruct((nbo, NP, CB), jnp.float32),
    )(agg, g, deg2, b, W)


def _fin(agg, g, deg2, b):
    nbi = agg.shape[0]
    d = nbi * CB
    return pl.pallas_call(
        _fin_body,
        grid=(NP // RB,),
        in_specs=[
            _blk3(nbi),
            _blk3(nbi),
            pl.BlockSpec((2, RB), lambda i: (0, i)),
            pl.BlockSpec((1, d), lambda i: (0, 0)),
        ],
        out_specs=pl.BlockSpec((RB, d), lambda i: (i, 0)),
        out_shape=jax.ShapeDtypeStruct((NP, d), jnp.float32),
    )(agg, g, deg2, b)


def kernel(x, edge_index, W1, b1, W2, b2, WL, bL):
    ei = edge_index.astype(jnp.int32)
    src, dst = ei[0], ei[1]
    xp = jnp.pad(x, ((0, NP - N), (0, 0)))
    deg2 = _deg_call(src, dst).reshape(NCORE, NDEG)[:, :NP]
    g1 = _mm1(xp, W1, deg2)
    agg1 = _scatter_4(src, dst, g1)
    g2 = _mid(agg1, g1, deg2, b1.reshape(1, -1), W2)
    agg2 = _scatter_4(src, dst, g2)
    g3 = _mid(agg2, g2, deg2, b2.reshape(1, -1), WL)
    agg3 = _scatter_2(src, dst, g3)
    out = _fin(agg3, g3, deg2, bL.reshape(1, -1))
    return out[:N]
